# baseline (speedup 1.0000x reference)
import functools

import jax
import jax.numpy as jnp
from jax import lax
from jax.experimental import pallas as pl
from jax.experimental.pallas import tpu as pltpu
from jax.experimental.pallas import tpu_sc as plsc

B = 16384
V = 100000
F = 128


_MV_SUB = 7168
_MV_NSTREAM = 2
_MV_BLK = _MV_SUB * _MV_NSTREAM
_MV_GRID = (V + _MV_BLK - 1) // _MV_BLK


def _mv_body(u_ref, *refs):
    o_ref = refs[-1]
    for k in range(_MV_NSTREAM):
        o_ref[pl.ds(k * _MV_SUB, _MV_SUB)] = lax.dot_general(
            u_ref[...], refs[k][...],
            dimension_numbers=(((1,), (1,)), ((), ())),
            preferred_element_type=jnp.float32,
        )[0]


def _matvec(user_table, item_table, row0, nrows):
    blk0 = row0 // _MV_SUB
    grid = (nrows + _MV_BLK - 1) // _MV_BLK

    def _stream_spec(k):
        return pl.BlockSpec((_MV_SUB, F),
                            lambda i, _k=k: (blk0 + i * _MV_NSTREAM + _k, 0))
    return pl.pallas_call(
        _mv_body,
        grid=(grid,),
        in_specs=[pl.BlockSpec((1, F), lambda i: (0, 0))] +
                 [_stream_spec(k) for k in range(_MV_NSTREAM)],
        out_specs=pl.BlockSpec((_MV_BLK,), lambda i: (i,)),
        out_shape=jax.ShapeDtypeStruct((nrows,), jnp.float32),
    )(user_table, *([item_table] * _MV_NSTREAM))


_NCc = 2
_NSc = 16
_SCV_RT = 1792
_SCV = _NCc * _NSc * _SCV_RT
_SCV_RC = 64
_SCV_NCH = _SCV_RT // _SCV_RC


def _shuf(x, idx):
    return lax.gather(
        x, idx[:, None],
        dimension_numbers=lax.GatherDimensionNumbers(
            offset_dims=(), collapsed_slice_dims=(0,), start_index_map=(0,)),
        slice_sizes=(1,),
        mode=lax.GatherScatterMode.PROMISE_IN_BOUNDS,
    )


def _sc_scores_body(item_hbm, user_hbm, out_hbm, u_v, bufa, bufb, dv,
                    sema, semb, semu):
    wid = lax.axis_index("s") * _NCc + lax.axis_index("c")
    rbase = wid * _SCV_RT
    pltpu.async_copy(user_hbm.at[0], u_v, semu).wait()
    us = [u_v[pl.ds(k * 16, 16)] for k in range(8)]
    bufs = (bufa, bufb)
    sems = (sema, semb)
    pltpu.async_copy(item_hbm.at[pl.ds(rbase, _SCV_RC)], bufa, sema)
    pltpu.async_copy(item_hbm.at[pl.ds(rbase + _SCV_RC, _SCV_RC)], bufb, semb)

    def body(c2, carry):
        for b in range(2):
            c = c2 * 2 + b
            row0 = rbase + c * _SCV_RC
            pltpu.make_async_copy(
                item_hbm.at[pl.ds(row0, _SCV_RC)], bufs[b], sems[b]).wait()
            lanes = lax.iota(jnp.int32, 16)
            perms = [lanes ^ s for s in (8, 4, 2, 1)]
            for g in range(_SCV_RC // 16):
                sv = jnp.zeros((16,), jnp.float32)
                for l in range(16):
                    r = g * 16 + l
                    acc = bufs[b][r, pl.ds(0, 16)] * us[0]
                    for k in range(1, 8):
                        acc = acc + bufs[b][r, pl.ds(k * 16, 16)] * us[k]
                    for p in perms:
                        acc = acc + _shuf(acc, p)
                    sv = jnp.where(lanes == l, acc, sv)
                dv[pl.ds(c * _SCV_RC + g * 16, 16)] = sv

            @pl.when(c + 2 < _SCV_NCH)
            def _():
                pltpu.async_copy(
                    item_hbm.at[pl.ds(row0 + 2 * _SCV_RC, _SCV_RC)],
                    bufs[b], sems[b])
        return carry

    lax.fori_loop(0, _SCV_NCH // 2, body, 0)
    pltpu.sync_copy(dv, out_hbm.at[pl.ds(rbase, _SCV_RT)])


def _sc_scores(user_table, item_table):
    mesh = plsc.VectorSubcoreMesh(core_axis_name="c", subcore_axis_name="s")
    kern = functools.partial(
        pl.kernel,
        out_type=jax.ShapeDtypeStruct((_SCV,), jnp.float32),
        mesh=mesh,
        scratch_types=[
            pltpu.VMEM((F,), jnp.float32),
            pltpu.VMEM((_SCV_RC, F), jnp.float32),
            pltpu.VMEM((_SCV_RC, F), jnp.float32),
            pltpu.VMEM((_SCV_RT,), jnp.float32),
            pltpu.SemaphoreType.DMA,
            pltpu.SemaphoreType.DMA,
            pltpu.SemaphoreType.DMA,
        ],
    )(_sc_scores_body)
    return kern(item_table, user_table)


_NC = 2
_NS = 16
_NW = _NC * _NS
_BPW = B // _NW
_CH = 128
_NCH = _BPW // _CH


def _sc_gather_body(scores_hbm, pos_hbm, neg_hbm, d_hbm,
                    idx_p, idx_n, vp, vn, dv, sem):
    wid = lax.axis_index("s") * _NC + lax.axis_index("c")
    base = wid * _BPW
    c1 = pltpu.async_copy(pos_hbm.at[pl.ds(base, _BPW)], idx_p, sem)
    c2 = pltpu.async_copy(neg_hbm.at[pl.ds(base, _BPW)], idx_n, sem)
    c1.wait()
    c2.wait()
    copies = []
    for j in range(_NCH):
        sl = pl.ds(j * _CH, _CH)
        copies.append(pltpu.async_copy(scores_hbm.at[idx_p.at[sl]], vp.at[sl], sem))
        copies.append(pltpu.async_copy(scores_hbm.at[idx_n.at[sl]], vn.at[sl], sem))
    for c in copies:
        c.wait()
    for i in range(_BPW // 16):
        sl = pl.ds(i * 16, 16)
        dv[sl] = vp[sl] - vn[sl]
    pltpu.sync_copy(dv, d_hbm.at[pl.ds(base, _BPW)])


def _sc_gather(scores, pos_i, neg_j):
    mesh = plsc.VectorSubcoreMesh(core_axis_name="c", subcore_axis_name="s")
    kern = functools.partial(
        pl.kernel,
        out_type=jax.ShapeDtypeStruct((B,), jnp.float32),
        mesh=mesh,
        scratch_types=[
            pltpu.VMEM((_BPW,), jnp.int32),
            pltpu.VMEM((_BPW,), jnp.int32),
            pltpu.VMEM((_BPW,), jnp.float32),
            pltpu.VMEM((_BPW,), jnp.float32),
            pltpu.VMEM((_BPW,), jnp.float32),
            pltpu.SemaphoreType.DMA,
        ],
    )(_sc_gather_body)
    return kern(scores, pos_i, neg_j)


def _loss_body(d_ref, o_ref):
    x = d_ref[...]
    ls = jnp.minimum(x, 0.0) - jnp.log1p(jnp.exp(-jnp.abs(x)))
    o_ref[0, 0] = -jnp.sum(ls)


def _loss(d2):
    return pl.pallas_call(
        _loss_body,
        out_specs=pl.BlockSpec(memory_space=pltpu.SMEM),
        out_shape=jax.ShapeDtypeStruct((1, 1), jnp.float32),
    )(d2)


def kernel(n_user, pos_i, neg_j, user_table, item_table):
    del n_user
    sc_part = _sc_scores(user_table, item_table)
    tc_part = _matvec(user_table, item_table, _SCV, V - _SCV)
    scores = jnp.concatenate([sc_part, tc_part])
    d = _sc_gather(scores, pos_i, neg_j)
    loss = _loss(d.reshape(128, 128))
    return loss[0, 0]


# speedup vs baseline: 1.1440x; 1.1440x over previous
import functools

import jax
import jax.numpy as jnp
from jax import lax
from jax.experimental import pallas as pl
from jax.experimental.pallas import tpu as pltpu
from jax.experimental.pallas import tpu_sc as plsc

B = 16384
V = 100000
F = 128


_MV_SUB = 7168
_MV_NSTREAM = 2
_MV_BLK = _MV_SUB * _MV_NSTREAM
_MV_GRID = (V + _MV_BLK - 1) // _MV_BLK


def _mv_body(u_ref, *refs):
    o_ref = refs[-1]
    for k in range(_MV_NSTREAM):
        o_ref[pl.ds(k * _MV_SUB, _MV_SUB)] = lax.dot_general(
            u_ref[...], refs[k][...],
            dimension_numbers=(((1,), (1,)), ((), ())),
            preferred_element_type=jnp.float32,
        )[0]


def _matvec(user_table, item_table, row0, nrows):
    blk0 = row0 // _MV_SUB
    grid = (nrows + _MV_BLK - 1) // _MV_BLK

    def _stream_spec(k):
        return pl.BlockSpec((_MV_SUB, F),
                            lambda i, _k=k: (blk0 + i * _MV_NSTREAM + _k, 0))
    return pl.pallas_call(
        _mv_body,
        grid=(grid,),
        in_specs=[pl.BlockSpec((1, F), lambda i: (0, 0))] +
                 [_stream_spec(k) for k in range(_MV_NSTREAM)],
        out_specs=pl.BlockSpec((_MV_BLK,), lambda i: (i,)),
        out_shape=jax.ShapeDtypeStruct((nrows,), jnp.float32),
    )(user_table, *([item_table] * _MV_NSTREAM))


_NCc = 2
_NSc = 16
_SCV_RT = 896
_SCV = _NCc * _NSc * _SCV_RT
_SCV_RC = 64
_SCV_NCH = _SCV_RT // _SCV_RC


def _shuf(x, idx):
    return lax.gather(
        x, idx[:, None],
        dimension_numbers=lax.GatherDimensionNumbers(
            offset_dims=(), collapsed_slice_dims=(0,), start_index_map=(0,)),
        slice_sizes=(1,),
        mode=lax.GatherScatterMode.PROMISE_IN_BOUNDS,
    )


def _sc_scores_body(item_hbm, user_hbm, out_hbm, u_v, bufa, bufb, dv, tmp,
                    sema, semb, semu):
    wid = lax.axis_index("s") * _NCc + lax.axis_index("c")
    rbase = wid * _SCV_RT
    pltpu.async_copy(user_hbm.at[0], u_v, semu).wait()
    us = [u_v[pl.ds(k * 16, 16)] for k in range(8)]
    bufs = (bufa, bufb)
    sems = (sema, semb)
    pltpu.async_copy(item_hbm.at[pl.ds(rbase, _SCV_RC)], bufa, sema)
    pltpu.async_copy(item_hbm.at[pl.ds(rbase + _SCV_RC, _SCV_RC)], bufb, semb)

    def body(c2, carry):
        for b in range(2):
            c = c2 * 2 + b
            row0 = rbase + c * _SCV_RC
            pltpu.make_async_copy(
                item_hbm.at[pl.ds(row0, _SCV_RC)], bufs[b], sems[b]).wait()
            lanes = lax.iota(jnp.int32, 16)
            perms = [lanes ^ s for s in (8, 4, 2, 1)]
            for g in range(_SCV_RC // 16):
                sv = jnp.zeros((16,), jnp.float32)
                for l in range(16):
                    r = g * 16 + l
                    acc = bufs[b][r, pl.ds(0, 16)] * us[0]
                    for k in range(1, 8):
                        acc = acc + bufs[b][r, pl.ds(k * 16, 16)] * us[k]
                    for p in perms:
                        acc = acc + _shuf(acc, p)
                    sv = jnp.where(lanes == l, acc, sv)
                dv[pl.ds(c * _SCV_RC + g * 16, 16)] = sv

            @pl.when(c + 2 < _SCV_NCH)
            def _():
                pltpu.async_copy(
                    item_hbm.at[pl.ds(row0 + 2 * _SCV_RC, _SCV_RC)],
                    bufs[b], sems[b])
        return carry

    lax.fori_loop(0, _SCV_NCH // 2, body, 0)
    pltpu.sync_copy(dv, out_hbm.at[pl.ds(rbase, _SCV_RT)])


def _sc_scores(user_table, item_table):
    mesh = plsc.VectorSubcoreMesh(core_axis_name="c", subcore_axis_name="s")
    kern = functools.partial(
        pl.kernel,
        out_type=jax.ShapeDtypeStruct((_SCV,), jnp.float32),
        mesh=mesh,
        scratch_types=[
            pltpu.VMEM((F,), jnp.float32),
            pltpu.VMEM((_SCV_RC, F), jnp.float32),
            pltpu.VMEM((_SCV_RC, F), jnp.float32),
            pltpu.VMEM((_SCV_RT,), jnp.float32),
            pltpu.VMEM((256,), jnp.float32),
            pltpu.SemaphoreType.DMA,
            pltpu.SemaphoreType.DMA,
            pltpu.SemaphoreType.DMA,
        ],
    )(_sc_scores_body)
    return kern(item_table, user_table)


_NC = 2
_NS = 16
_NW = _NC * _NS
_BPW = B // _NW
_CH = 128
_NCH = _BPW // _CH


def _sc_gather_body(scores_hbm, pos_hbm, neg_hbm, d_hbm,
                    idx_p, idx_n, vp, vn, dv, sem):
    wid = lax.axis_index("s") * _NC + lax.axis_index("c")
    base = wid * _BPW
    c1 = pltpu.async_copy(pos_hbm.at[pl.ds(base, _BPW)], idx_p, sem)
    c2 = pltpu.async_copy(neg_hbm.at[pl.ds(base, _BPW)], idx_n, sem)
    c1.wait()
    c2.wait()
    copies = []
    for j in range(_NCH):
        sl = pl.ds(j * _CH, _CH)
        copies.append(pltpu.async_copy(scores_hbm.at[idx_p.at[sl]], vp.at[sl], sem))
        copies.append(pltpu.async_copy(scores_hbm.at[idx_n.at[sl]], vn.at[sl], sem))
    for c in copies:
        c.wait()
    for i in range(_BPW // 16):
        sl = pl.ds(i * 16, 16)
        dv[sl] = vp[sl] - vn[sl]
    pltpu.sync_copy(dv, d_hbm.at[pl.ds(base, _BPW)])


def _sc_gather(scores, pos_i, neg_j):
    mesh = plsc.VectorSubcoreMesh(core_axis_name="c", subcore_axis_name="s")
    kern = functools.partial(
        pl.kernel,
        out_type=jax.ShapeDtypeStruct((B,), jnp.float32),
        mesh=mesh,
        scratch_types=[
            pltpu.VMEM((_BPW,), jnp.int32),
            pltpu.VMEM((_BPW,), jnp.int32),
            pltpu.VMEM((_BPW,), jnp.float32),
            pltpu.VMEM((_BPW,), jnp.float32),
            pltpu.VMEM((_BPW,), jnp.float32),
            pltpu.SemaphoreType.DMA,
        ],
    )(_sc_gather_body)
    return kern(scores, pos_i, neg_j)


def _loss_body(d_ref, o_ref):
    x = d_ref[...]
    ls = jnp.minimum(x, 0.0) - jnp.log1p(jnp.exp(-jnp.abs(x)))
    o_ref[0, 0] = -jnp.sum(ls)


def _loss(d2):
    return pl.pallas_call(
        _loss_body,
        out_specs=pl.BlockSpec(memory_space=pltpu.SMEM),
        out_shape=jax.ShapeDtypeStruct((1, 1), jnp.float32),
    )(d2)


def kernel(n_user, pos_i, neg_j, user_table, item_table):
    del n_user
    sc_part = _sc_scores(user_table, item_table)
    tc_part = _matvec(user_table, item_table, _SCV, V - _SCV)
    scores = jnp.concatenate([sc_part, tc_part])
    d = _sc_gather(scores, pos_i, neg_j)
    loss = _loss(d.reshape(128, 128))
    return loss[0, 0]


# speedup vs baseline: 1.3593x; 1.1882x over previous
import functools

import jax
import jax.numpy as jnp
from jax import lax
from jax.experimental import pallas as pl
from jax.experimental.pallas import tpu as pltpu
from jax.experimental.pallas import tpu_sc as plsc

B = 16384
V = 100000
F = 128


_MV_SUB = 7168
_MV_NSTREAM = 2
_MV_BLK = _MV_SUB * _MV_NSTREAM
_MV_GRID = (V + _MV_BLK - 1) // _MV_BLK


def _mv_body(u_ref, *refs):
    o_ref = refs[-1]
    for k in range(_MV_NSTREAM):
        o_ref[pl.ds(k * _MV_SUB, _MV_SUB)] = lax.dot_general(
            u_ref[...], refs[k][...],
            dimension_numbers=(((1,), (1,)), ((), ())),
            preferred_element_type=jnp.float32,
        )[0]


def _matvec(user_table, item_table, row0, nrows):
    blk0 = row0 // _MV_SUB
    grid = (nrows + _MV_BLK - 1) // _MV_BLK

    def _stream_spec(k):
        return pl.BlockSpec((_MV_SUB, F),
                            lambda i, _k=k: (blk0 + i * _MV_NSTREAM + _k, 0))
    return pl.pallas_call(
        _mv_body,
        grid=(grid,),
        in_specs=[pl.BlockSpec((1, F), lambda i: (0, 0))] +
                 [_stream_spec(k) for k in range(_MV_NSTREAM)],
        out_specs=pl.BlockSpec((_MV_BLK,), lambda i: (i,)),
        out_shape=jax.ShapeDtypeStruct((nrows,), jnp.float32),
    )(user_table, *([item_table] * _MV_NSTREAM))


_NC = 2
_NS = 16
_NW = _NC * _NS
_BPW = B // _NW
_CH = 128
_NCH = _BPW // _CH


def _sc_gather_body(scores_hbm, pos_hbm, neg_hbm, d_hbm,
                    idx_p, idx_n, vp, vn, dv, sem):
    wid = lax.axis_index("s") * _NC + lax.axis_index("c")
    base = wid * _BPW
    c1 = pltpu.async_copy(pos_hbm.at[pl.ds(base, _BPW)], idx_p, sem)
    c2 = pltpu.async_copy(neg_hbm.at[pl.ds(base, _BPW)], idx_n, sem)
    c1.wait()
    c2.wait()
    copies = []
    for j in range(_NCH):
        sl = pl.ds(j * _CH, _CH)
        copies.append(pltpu.async_copy(scores_hbm.at[idx_p.at[sl]], vp.at[sl], sem))
        copies.append(pltpu.async_copy(scores_hbm.at[idx_n.at[sl]], vn.at[sl], sem))
    for c in copies:
        c.wait()
    for i in range(_BPW // 16):
        sl = pl.ds(i * 16, 16)
        dv[sl] = vp[sl] - vn[sl]
    pltpu.sync_copy(dv, d_hbm.at[pl.ds(base, _BPW)])


def _sc_gather(scores, pos_i, neg_j):
    mesh = plsc.VectorSubcoreMesh(core_axis_name="c", subcore_axis_name="s")
    kern = functools.partial(
        pl.kernel,
        out_type=jax.ShapeDtypeStruct((B,), jnp.float32),
        mesh=mesh,
        scratch_types=[
            pltpu.VMEM((_BPW,), jnp.int32),
            pltpu.VMEM((_BPW,), jnp.int32),
            pltpu.VMEM((_BPW,), jnp.float32),
            pltpu.VMEM((_BPW,), jnp.float32),
            pltpu.VMEM((_BPW,), jnp.float32),
            pltpu.SemaphoreType.DMA,
        ],
    )(_sc_gather_body)
    return kern(scores, pos_i, neg_j)


def _loss_body(d_ref, o_ref):
    x = d_ref[...]
    ls = jnp.minimum(x, 0.0) - jnp.log1p(jnp.exp(-jnp.abs(x)))
    o_ref[0, 0] = -jnp.sum(ls)


def _loss(d2):
    return pl.pallas_call(
        _loss_body,
        out_specs=pl.BlockSpec(memory_space=pltpu.SMEM),
        out_shape=jax.ShapeDtypeStruct((1, 1), jnp.float32),
    )(d2)


def kernel(n_user, pos_i, neg_j, user_table, item_table):
    del n_user
    scores = _matvec(user_table, item_table, 0, V)
    d = _sc_gather(scores, pos_i, neg_j)
    loss = _loss(d.reshape(128, 128))
    return loss[0, 0]
